# Initial kernel scaffold; baseline (speedup 1.0000x reference)
import jax
import jax.numpy as jnp
from jax.experimental import pallas as pl


def kernel(target_word, context_word, negative_example, target_table, context_table):
    raise NotImplementedError("write your pallas kernel here")


# speedup vs baseline: 4.8883x; 4.8883x over previous
import functools

import jax
import jax.numpy as jnp
from jax import lax
from jax.experimental import pallas as pl
from jax.experimental.pallas import tpu as pltpu
from jax.experimental.pallas import tpu_sc as plsc

VOCAB = 1000000
EMB = 64
B = 16384
K_NEG = 20

NC, NS, L = 2, 16, 16
NW = NC * NS
BPW = B // NW
CB = 16
NCH = BPW // CB
NEG_PER_CHUNK = CB * K_NEG


def _sc_dots(tw_hbm, cw_hbm, ne_hbm, ttab_hbm, ctab_hbm, pos_hbm, neg_hbm,
             tidx, cidx, nidx, trow, crow, nrow, ptmp, ntmp, pos_v, neg_v, sem):
    wid = lax.axis_index("s") * NC + lax.axis_index("c")
    base = wid * BPW
    pltpu.sync_copy(tw_hbm.at[pl.ds(base, BPW)], tidx)
    pltpu.sync_copy(cw_hbm.at[pl.ds(base, BPW)], cidx)
    pltpu.sync_copy(ne_hbm.at[pl.ds(wid * (BPW * K_NEG), BPW * K_NEG)], nidx)

    @pl.loop(0, NCH)
    def _chunk(g):
        off = g * CB
        descs = [
            pltpu.async_copy(ttab_hbm.at[tidx.at[pl.ds(off, CB)]], trow, sem),
            pltpu.async_copy(ctab_hbm.at[cidx.at[pl.ds(off, CB)]], crow, sem),
        ]
        for q in range(NEG_PER_CHUNK // 64):
            descs.append(pltpu.async_copy(
                ctab_hbm.at[nidx.at[pl.ds(g * NEG_PER_CHUNK + q * 64, 64)]],
                nrow.at[pl.ds(q * 64, 64)], sem))
        for d in descs:
            d.wait()
        for b in range(CB):
            t = [trow[b, pl.ds(16 * j, 16)] for j in range(4)]
            c = [crow[b, pl.ds(16 * j, 16)] for j in range(4)]
            p = t[0] * c[0] + t[1] * c[1] + t[2] * c[2] + t[3] * c[3]
            acc = [nrow[b * K_NEG, pl.ds(16 * j, 16)] for j in range(4)]
            for k in range(1, K_NEG):
                for j in range(4):
                    acc[j] = acc[j] + nrow[b * K_NEG + k, pl.ds(16 * j, 16)]
            nd = acc[0] * t[0] + acc[1] * t[1] + acc[2] * t[2] + acc[3] * t[3]
            ptmp[b, pl.ds(0, 16)] = p
            ntmp[b, pl.ds(0, 16)] = nd
        lane = lax.iota(jnp.int32, 16)
        posvec = jnp.zeros((16,), jnp.float32)
        negvec = jnp.zeros((16,), jnp.float32)
        for col in range(16):
            cc = jnp.full((16,), col, jnp.int32)
            posvec = posvec + plsc.load_gather(ptmp, [lane, cc])
            negvec = negvec + plsc.load_gather(ntmp, [lane, cc])
        pos_v[pl.ds(off, CB)] = posvec
        neg_v[pl.ds(off, CB)] = negvec

    pltpu.sync_copy(pos_v, pos_hbm.at[pl.ds(base, BPW)])
    pltpu.sync_copy(neg_v, neg_hbm.at[pl.ds(base, BPW)])


def _loss_body(pos_ref, neg_ref, o_ref):
    p = pos_ref[...]
    n = -neg_ref[...]
    lsp = jnp.minimum(p, 0.0) - jnp.log1p(jnp.exp(-jnp.abs(p)))
    lsn = jnp.minimum(n, 0.0) - jnp.log1p(jnp.exp(-jnp.abs(n)))
    o_ref[0, 0] = -(jnp.sum(lsp) + jnp.sum(lsn))


def kernel(target_word, context_word, negative_example, target_table, context_table):
    tw = target_word.astype(jnp.int32)
    cw = context_word.astype(jnp.int32)
    ne = negative_example.astype(jnp.int32).reshape(B * K_NEG)

    mesh = plsc.VectorSubcoreMesh(
        core_axis_name="c", subcore_axis_name="s",
        num_cores=NC, num_subcores=NS)
    pos, neg = pl.kernel(
        _sc_dots,
        out_type=(
            jax.ShapeDtypeStruct((B,), jnp.float32),
            jax.ShapeDtypeStruct((B,), jnp.float32),
        ),
        mesh=mesh,
        compiler_params=pltpu.CompilerParams(
            needs_layout_passes=False, use_tc_tiling_on_sc=False),
        scratch_types=[
            pltpu.VMEM((BPW,), jnp.int32),
            pltpu.VMEM((BPW,), jnp.int32),
            pltpu.VMEM((BPW * K_NEG,), jnp.int32),
            pltpu.VMEM((CB, EMB), jnp.float32),
            pltpu.VMEM((CB, EMB), jnp.float32),
            pltpu.VMEM((NEG_PER_CHUNK, EMB), jnp.float32),
            pltpu.VMEM((CB, 16), jnp.float32),
            pltpu.VMEM((CB, 16), jnp.float32),
            pltpu.VMEM((BPW,), jnp.float32),
            pltpu.VMEM((BPW,), jnp.float32),
            pltpu.SemaphoreType.DMA,
        ],
    )(tw, cw, ne, target_table, context_table)

    loss = pl.pallas_call(
        _loss_body,
        out_shape=jax.ShapeDtypeStruct((1, 1), jnp.float32),
        in_specs=[
            pl.BlockSpec(memory_space=pltpu.VMEM),
            pl.BlockSpec(memory_space=pltpu.VMEM),
        ],
        out_specs=pl.BlockSpec(memory_space=pltpu.SMEM),
    )(pos.reshape(128, 128), neg.reshape(128, 128))
    return loss[0, 0]


# speedup vs baseline: 4.9903x; 1.0209x over previous
import jax
import jax.numpy as jnp
from jax import lax
from jax.experimental import pallas as pl
from jax.experimental.pallas import tpu as pltpu
from jax.experimental.pallas import tpu_sc as plsc

VOCAB = 1000000
EMB = 64
B = 16384
K_NEG = 20

NC, NS, L = 2, 16, 16
NW = NC * NS
BPW = B // NW
CB = 16
NCH = BPW // CB
NPC = CB * K_NEG
NIX = BPW * K_NEG


def _sc_dots(tw_hbm, cw_hbm, ne_hbm, ttab_hbm, ctab_hbm, pos_hbm, neg_hbm,
             tidx, cidx, nidx, tpair, cpair, npair,
             trow, crow, nrow, ptmp, ntmp, pos_v, neg_v, sem):
    wid = lax.axis_index("s") * NC + lax.axis_index("c")
    base = wid * BPW
    pltpu.sync_copy(tw_hbm.at[pl.ds(base, BPW)], tidx)
    pltpu.sync_copy(cw_hbm.at[pl.ds(base, BPW)], cidx)
    pltpu.sync_copy(ne_hbm.at[pl.ds(wid * NIX, NIX)], nidx)
    for i in range(BPW // 16):
        tpair[pl.ds(16 * i, 16)] = tidx[pl.ds(16 * i, 16)] >> 1
        cpair[pl.ds(16 * i, 16)] = cidx[pl.ds(16 * i, 16)] >> 1
    for i in range(NIX // 16):
        npair[pl.ds(16 * i, 16)] = nidx[pl.ds(16 * i, 16)] >> 1

    @pl.loop(0, NCH)
    def _chunk(g):
        off = g * CB
        descs = [
            pltpu.async_copy(ttab_hbm.at[tpair.at[pl.ds(off, CB)]], trow, sem),
            pltpu.async_copy(ctab_hbm.at[cpair.at[pl.ds(off, CB)]], crow, sem),
        ]
        for q in range(NPC // 64):
            descs.append(pltpu.async_copy(
                ctab_hbm.at[npair.at[pl.ds(g * NPC + q * 64, 64)]],
                nrow.at[pl.ds(q * 64, 64)], sem))
        for d in descs:
            d.wait()
        tho = (tidx[pl.ds(off, CB)] & 1) * 64
        cho = (cidx[pl.ds(off, CB)] & 1) * 64
        nho = [(nidx[pl.ds(g * NPC + 16 * i, 16)] & 1) * 64
               for i in range(NPC // 16)]
        for b in range(CB):
            th = tho[b]
            ch = cho[b]
            t = [trow[b, pl.ds(th + 16 * j, 16)] for j in range(4)]
            c = [crow[b, pl.ds(ch + 16 * j, 16)] for j in range(4)]
            p = t[0] * c[0] + t[1] * c[1] + t[2] * c[2] + t[3] * c[3]
            m = b * K_NEG
            nh = nho[m // 16][m % 16]
            acc = [nrow[m, pl.ds(nh + 16 * j, 16)] for j in range(4)]
            for k in range(1, K_NEG):
                m = b * K_NEG + k
                nh = nho[m // 16][m % 16]
                for j in range(4):
                    acc[j] = acc[j] + nrow[m, pl.ds(nh + 16 * j, 16)]
            nd = acc[0] * t[0] + acc[1] * t[1] + acc[2] * t[2] + acc[3] * t[3]
            ptmp[b, pl.ds(0, 16)] = p
            ntmp[b, pl.ds(0, 16)] = nd
        lane = lax.iota(jnp.int32, 16)
        posvec = jnp.zeros((16,), jnp.float32)
        negvec = jnp.zeros((16,), jnp.float32)
        for col in range(16):
            cc = jnp.full((16,), col, jnp.int32)
            posvec = posvec + plsc.load_gather(ptmp, [lane, cc])
            negvec = negvec + plsc.load_gather(ntmp, [lane, cc])
        pos_v[pl.ds(off, CB)] = posvec
        neg_v[pl.ds(off, CB)] = negvec

    pltpu.sync_copy(pos_v, pos_hbm.at[pl.ds(base, BPW)])
    pltpu.sync_copy(neg_v, neg_hbm.at[pl.ds(base, BPW)])


def _loss_body(pos_ref, neg_ref, o_ref):
    p = pos_ref[...]
    n = -neg_ref[...]
    lsp = jnp.minimum(p, 0.0) - jnp.log1p(jnp.exp(-jnp.abs(p)))
    lsn = jnp.minimum(n, 0.0) - jnp.log1p(jnp.exp(-jnp.abs(n)))
    o_ref[0, 0] = -(jnp.sum(lsp) + jnp.sum(lsn))


def kernel(target_word, context_word, negative_example, target_table, context_table):
    tw = target_word.astype(jnp.int32)
    cw = context_word.astype(jnp.int32)
    ne = negative_example.astype(jnp.int32).reshape(B * K_NEG)
    ttab2 = target_table.reshape(VOCAB // 2, 2 * EMB)
    ctab2 = context_table.reshape(VOCAB // 2, 2 * EMB)

    mesh = plsc.VectorSubcoreMesh(
        core_axis_name="c", subcore_axis_name="s",
        num_cores=NC, num_subcores=NS)
    pos, neg = pl.kernel(
        _sc_dots,
        out_type=(
            jax.ShapeDtypeStruct((B,), jnp.float32),
            jax.ShapeDtypeStruct((B,), jnp.float32),
        ),
        mesh=mesh,
        compiler_params=pltpu.CompilerParams(
            needs_layout_passes=False, use_tc_tiling_on_sc=True),
        scratch_types=[
            pltpu.VMEM((BPW,), jnp.int32),
            pltpu.VMEM((BPW,), jnp.int32),
            pltpu.VMEM((NIX,), jnp.int32),
            pltpu.VMEM((BPW,), jnp.int32),
            pltpu.VMEM((BPW,), jnp.int32),
            pltpu.VMEM((NIX,), jnp.int32),
            pltpu.VMEM((CB, 2 * EMB), jnp.float32),
            pltpu.VMEM((CB, 2 * EMB), jnp.float32),
            pltpu.VMEM((NPC, 2 * EMB), jnp.float32),
            pltpu.VMEM((CB, 16), jnp.float32),
            pltpu.VMEM((CB, 16), jnp.float32),
            pltpu.VMEM((BPW,), jnp.float32),
            pltpu.VMEM((BPW,), jnp.float32),
            pltpu.SemaphoreType.DMA,
        ],
    )(tw, cw, ne, ttab2, ctab2)

    loss = pl.pallas_call(
        _loss_body,
        out_shape=jax.ShapeDtypeStruct((1, 1), jnp.float32),
        in_specs=[
            pl.BlockSpec(memory_space=pltpu.VMEM),
            pl.BlockSpec(memory_space=pltpu.VMEM),
        ],
        out_specs=pl.BlockSpec(memory_space=pltpu.SMEM),
    )(pos.reshape(128, 128), neg.reshape(128, 128))
    return loss[0, 0]
